# baseline (speedup 1.0000x reference)
import functools

import jax
import jax.numpy as jnp
from jax import lax
from jax.experimental import pallas as pl
from jax.experimental.pallas import tpu as pltpu
from jax.experimental.pallas import tpu_sc as plsc

_ROT_A = (13, 15, 26, 6)
_ROT_B = (17, 29, 16, 24)
_BIG = 1e10


def _rotl(x, r):
    return (x << jnp.uint32(r)) | (x >> jnp.uint32(32 - r))


def _rounds(x0, x1, rots):
    for r in rots:
        x0 = x0 + x1
        x1 = _rotl(x1, r)
        x1 = x0 ^ x1
    return x0, x1


def _threefry_bits(k0, k1, x1):
    ks2 = k0 ^ k1 ^ jnp.uint32(0x1BD11BDA)
    x0 = x1 + k0
    x1 = _rotl(x1, _ROT_A[0])
    x1 = x0 ^ x1
    x0, x1 = _rounds(x0, x1, _ROT_A[1:])
    x0 = x0 + k1
    x1 = x1 + (ks2 + jnp.uint32(1))
    x0, x1 = _rounds(x0, x1, _ROT_B)
    x0 = x0 + ks2
    x1 = x1 + (k0 + jnp.uint32(2))
    x0, x1 = _rounds(x0, x1, _ROT_A)
    x0 = x0 + k0
    x1 = x1 + (k1 + jnp.uint32(3))
    x0, x1 = _rounds(x0, x1, _ROT_B)
    x0 = x0 + k1
    x1 = x1 + (ks2 + jnp.uint32(4))
    x0, x1 = _rounds(x0, x1, _ROT_A)
    x0 = x0 + ks2
    x1 = x1 + (k0 + jnp.uint32(5))
    return x0 ^ x1


def _gu_tile(logits_ref, gumbel_ref, k0, k1, rowk, c0, tl, rb):
    col_u = lax.broadcasted_iota(jnp.uint32, (rb, tl), 1)
    x1 = col_u + (rowk + jnp.uint32(c0) if isinstance(c0, int)
                  else rowk + c0.astype(jnp.uint32))
    bits = _threefry_bits(k0, k1, x1)
    fb = (bits >> jnp.uint32(9)) | jnp.uint32(0x3F800000)
    f = lax.bitcast_convert_type(fb, jnp.float32)
    e = -jnp.log(2.0 - f)
    gum = jnp.minimum(-jnp.log(e), _BIG)
    gu = logits_ref[:, pl.ds(c0, tl)] + gum
    gumbel_ref[:, pl.ds(c0, tl)] = gu
    return gu


_TILE = 4096


def _rebar_body(key_ref, logits_ref, gumbel_ref, hard_ref, idx_ref, *, rb, v):
    g = pl.program_id(0)
    k0 = key_ref[0]
    k1 = key_ref[1]
    nt = v // _TILE
    tail = v - nt * _TILE
    c_tail = nt * _TILE
    rowk = (
        lax.broadcasted_iota(jnp.uint32, (rb, 1), 0) * jnp.uint32(v)
        + ((g * (rb * v)).astype(jnp.uint32) + k1)
    )

    def step_a(j, macc):
        c0 = pl.multiple_of(j * _TILE, _TILE)
        gu = _gu_tile(logits_ref, gumbel_ref, k0, k1, rowk, c0, _TILE, rb)
        return jnp.maximum(macc, gu)

    macc = lax.fori_loop(
        0, nt, step_a, jnp.full((rb, _TILE), -jnp.inf, jnp.float32)
    )
    gu_t = _gu_tile(logits_ref, gumbel_ref, k0, k1, rowk, c_tail, tail, rb)
    m = jnp.maximum(
        jnp.max(macc, axis=1, keepdims=True),
        jnp.max(gu_t, axis=1, keepdims=True),
    )

    def step_b(j, carry):
        sacc, iacc = carry
        c0 = pl.multiple_of(j * _TILE, _TILE)
        gu = gumbel_ref[:, pl.ds(c0, _TILE)]
        sacc = sacc + jnp.exp(gu - m)
        col = lax.broadcasted_iota(jnp.int32, (rb, _TILE), 1) + c0
        iacc = jnp.minimum(iacc, jnp.where(gu >= m, col, v))
        return sacc, iacc

    sacc, iacc = lax.fori_loop(
        0,
        nt,
        step_b,
        (
            jnp.zeros((rb, _TILE), jnp.float32),
            jnp.full((rb, _TILE), v, jnp.int32),
        ),
    )
    p_t = jnp.exp(gu_t - m)
    col_t = lax.broadcasted_iota(jnp.int32, (rb, tail), 1) + c_tail
    s = (
        jnp.sum(sacc, axis=1, keepdims=True)
        + jnp.sum(p_t, axis=1, keepdims=True)
    )
    idx = jnp.minimum(
        jnp.min(iacc, axis=1, keepdims=True),
        jnp.min(jnp.where(gu_t >= m, col_t, v), axis=1, keepdims=True),
    )
    d = m + jnp.log(s)

    def step_c(j, carry):
        c0 = pl.multiple_of(j * _TILE, _TILE)
        gu = gumbel_ref[:, pl.ds(c0, _TILE)]
        gumbel_ref[:, pl.ds(c0, _TILE)] = jnp.exp(gu - d)
        col = lax.broadcasted_iota(jnp.int32, (rb, _TILE), 1) + c0
        hard_ref[:, pl.ds(c0, _TILE)] = jnp.where(col == idx, 1.0, 0.0)
        return carry

    lax.fori_loop(0, nt, step_c, 0)
    gumbel_ref[:, pl.ds(c_tail, tail)] = jnp.exp(gu_t - d)
    hard_ref[:, pl.ds(c_tail, tail)] = jnp.where(col_t == idx, 1.0, 0.0)
    idx_ref[...] = jnp.broadcast_to(idx, (rb, 128))


def _dense_stage(key_data, logits2, rb):
    r, v = logits2.shape
    return pl.pallas_call(
        functools.partial(_rebar_body, rb=rb, v=v),
        grid=(r // rb,),
        in_specs=[
            pl.BlockSpec(memory_space=pltpu.SMEM),
            pl.BlockSpec((rb, v), lambda g: (g, 0)),
        ],
        out_specs=[
            pl.BlockSpec((rb, v), lambda g: (g, 0)),
            pl.BlockSpec((rb, v), lambda g: (g, 0)),
            pl.BlockSpec((rb, 128), lambda g: (g, 0)),
        ],
        out_shape=[
            jax.ShapeDtypeStruct((r, v), jnp.float32),
            jax.ShapeDtypeStruct((r, v), jnp.float32),
            jax.ShapeDtypeStruct((r, 128), jnp.int32),
        ],
        compiler_params=pltpu.CompilerParams(
            dimension_semantics=("parallel",),
        ),
    )(key_data, logits2)


def _sc_gather(table, idx):
    r = idx.shape[0]
    d = table.shape[1]
    info = plsc.get_sparse_core_info()
    nc = info.num_cores
    nw_used = 16
    rows_per = r // nw_used
    mesh = plsc.VectorSubcoreMesh(core_axis_name="c", subcore_axis_name="s")

    @functools.partial(
        pl.kernel,
        mesh=mesh,
        out_type=jax.ShapeDtypeStruct((r, d), jnp.float32),
        scratch_types=[
            pltpu.VMEM((rows_per,), jnp.int32),
            pltpu.VMEM((rows_per, d), jnp.float32),
            pltpu.SemaphoreType.DMA,
        ],
    )
    def gk(table_hbm, idx_hbm, out_hbm, idx_v, rows_v, sem):
        wid = lax.axis_index("s") * nc + lax.axis_index("c")

        @pl.when(wid < nw_used)
        def _():
            base = wid * rows_per
            pltpu.sync_copy(idx_hbm.at[pl.ds(base, rows_per)], idx_v)
            pltpu.async_copy(table_hbm.at[idx_v], rows_v, sem).wait()
            pltpu.sync_copy(rows_v, out_hbm.at[pl.ds(base, rows_per)])

    return gk(table, idx)


def kernel(logits, emb, sent_mask, f_mean, g_mean, g2_mean, fg_mean):
    b, s, v = logits.shape
    d = emb.shape[1]
    r = b * s
    key_data = jax.random.key_data(
        jax.random.split(jax.random.key(42), 8)
    )[0].astype(jnp.uint32)
    gumbel2, hard2, idxb = _dense_stage(key_data, logits.reshape(r, v), rb=16)
    idx_flat = idxb[:, 0]
    f = _sc_gather(emb, idx_flat).reshape(b, s, d)
    return f, gumbel2.reshape(b, s, v), hard2.reshape(b, s, v)


# speedup vs baseline: 1.0166x; 1.0166x over previous
import functools

import jax
import jax.numpy as jnp
from jax import lax
from jax.experimental import pallas as pl
from jax.experimental.pallas import tpu as pltpu
from jax.experimental.pallas import tpu_sc as plsc

_ROT_A = (13, 15, 26, 6)
_ROT_B = (17, 29, 16, 24)
_BIG = 1e10


def _rotl(x, r):
    return (x << jnp.uint32(r)) | (x >> jnp.uint32(32 - r))


def _rounds(x0, x1, rots):
    for r in rots:
        x0 = x0 + x1
        x1 = _rotl(x1, r)
        x1 = x0 ^ x1
    return x0, x1


def _threefry_bits(k0, k1, x1):
    ks2 = k0 ^ k1 ^ jnp.uint32(0x1BD11BDA)
    x0 = x1 + k0
    x1 = _rotl(x1, _ROT_A[0])
    x1 = x0 ^ x1
    x0, x1 = _rounds(x0, x1, _ROT_A[1:])
    x0 = x0 + k1
    x1 = x1 + (ks2 + jnp.uint32(1))
    x0, x1 = _rounds(x0, x1, _ROT_B)
    x0 = x0 + ks2
    x1 = x1 + (k0 + jnp.uint32(2))
    x0, x1 = _rounds(x0, x1, _ROT_A)
    x0 = x0 + k0
    x1 = x1 + (k1 + jnp.uint32(3))
    x0, x1 = _rounds(x0, x1, _ROT_B)
    x0 = x0 + k1
    x1 = x1 + (ks2 + jnp.uint32(4))
    x0, x1 = _rounds(x0, x1, _ROT_A)
    x0 = x0 + ks2
    x1 = x1 + (k0 + jnp.uint32(5))
    return x0 ^ x1


def _gu_tile(logits_ref, gumbel_ref, k0, k1, rowk, c0, tl, rb):
    col_u = lax.broadcasted_iota(jnp.uint32, (rb, tl), 1)
    x1 = col_u + (rowk + jnp.uint32(c0) if isinstance(c0, int)
                  else rowk + c0.astype(jnp.uint32))
    bits = _threefry_bits(k0, k1, x1)
    fb = (bits >> jnp.uint32(9)) | jnp.uint32(0x3F800000)
    f = lax.bitcast_convert_type(fb, jnp.float32)
    e = -jnp.log(2.0 - f)
    gum = jnp.minimum(-jnp.log(e), _BIG)
    gu = logits_ref[:, pl.ds(c0, tl)] + gum
    gumbel_ref[:, pl.ds(c0, tl)] = gu
    return gu


_TILE = 8192


def _rebar_body(key_ref, logits_ref, gumbel_ref, hard_ref, idx_ref, *, rb, v):
    g = pl.program_id(0)
    k0 = key_ref[0]
    k1 = key_ref[1]
    nt = v // _TILE
    tail = v - nt * _TILE
    c_tail = nt * _TILE
    rowk = (
        lax.broadcasted_iota(jnp.uint32, (rb, 1), 0) * jnp.uint32(v)
        + ((g * (rb * v)).astype(jnp.uint32) + k1)
    )

    def step_a(j, macc):
        c0 = pl.multiple_of(j * _TILE, _TILE)
        gu = _gu_tile(logits_ref, gumbel_ref, k0, k1, rowk, c0, _TILE, rb)
        return jnp.maximum(macc, gu)

    macc = lax.fori_loop(
        0, nt, step_a, jnp.full((rb, _TILE), -jnp.inf, jnp.float32),
        unroll=2,
    )
    gu_t = _gu_tile(logits_ref, gumbel_ref, k0, k1, rowk, c_tail, tail, rb)
    m = jnp.maximum(
        jnp.max(macc, axis=1, keepdims=True),
        jnp.max(gu_t, axis=1, keepdims=True),
    )

    def step_b(j, carry):
        sacc, iacc = carry
        c0 = pl.multiple_of(j * _TILE, _TILE)
        gu = gumbel_ref[:, pl.ds(c0, _TILE)]
        sacc = sacc + jnp.exp(gu - m)
        col = lax.broadcasted_iota(jnp.int32, (rb, _TILE), 1) + c0
        iacc = jnp.minimum(iacc, jnp.where(gu >= m, col, v))
        return sacc, iacc

    sacc, iacc = lax.fori_loop(
        0,
        nt,
        step_b,
        (
            jnp.zeros((rb, _TILE), jnp.float32),
            jnp.full((rb, _TILE), v, jnp.int32),
        ),
    )
    p_t = jnp.exp(gu_t - m)
    col_t = lax.broadcasted_iota(jnp.int32, (rb, tail), 1) + c_tail
    s = (
        jnp.sum(sacc, axis=1, keepdims=True)
        + jnp.sum(p_t, axis=1, keepdims=True)
    )
    idx = jnp.minimum(
        jnp.min(iacc, axis=1, keepdims=True),
        jnp.min(jnp.where(gu_t >= m, col_t, v), axis=1, keepdims=True),
    )
    d = m + jnp.log(s)

    def step_c(j, carry):
        c0 = pl.multiple_of(j * _TILE, _TILE)
        gu = gumbel_ref[:, pl.ds(c0, _TILE)]
        gumbel_ref[:, pl.ds(c0, _TILE)] = jnp.exp(gu - d)
        col = lax.broadcasted_iota(jnp.int32, (rb, _TILE), 1) + c0
        hard_ref[:, pl.ds(c0, _TILE)] = jnp.where(col == idx, 1.0, 0.0)
        return carry

    lax.fori_loop(0, nt, step_c, 0)
    gumbel_ref[:, pl.ds(c_tail, tail)] = jnp.exp(gu_t - d)
    hard_ref[:, pl.ds(c_tail, tail)] = jnp.where(col_t == idx, 1.0, 0.0)
    idx_ref[...] = jnp.broadcast_to(idx, (rb, 128))


def _dense_stage(key_data, logits2, rb):
    r, v = logits2.shape
    return pl.pallas_call(
        functools.partial(_rebar_body, rb=rb, v=v),
        grid=(r // rb,),
        in_specs=[
            pl.BlockSpec(memory_space=pltpu.SMEM),
            pl.BlockSpec((rb, v), lambda g: (g, 0)),
        ],
        out_specs=[
            pl.BlockSpec((rb, v), lambda g: (g, 0)),
            pl.BlockSpec((rb, v), lambda g: (g, 0)),
            pl.BlockSpec((rb, 128), lambda g: (g, 0)),
        ],
        out_shape=[
            jax.ShapeDtypeStruct((r, v), jnp.float32),
            jax.ShapeDtypeStruct((r, v), jnp.float32),
            jax.ShapeDtypeStruct((r, 128), jnp.int32),
        ],
        compiler_params=pltpu.CompilerParams(
            dimension_semantics=("parallel",),
        ),
    )(key_data, logits2)


def _sc_gather(table, idx):
    r = idx.shape[0]
    d = table.shape[1]
    info = plsc.get_sparse_core_info()
    nc = info.num_cores
    nw_used = 16
    rows_per = r // nw_used
    mesh = plsc.VectorSubcoreMesh(core_axis_name="c", subcore_axis_name="s")

    @functools.partial(
        pl.kernel,
        mesh=mesh,
        out_type=jax.ShapeDtypeStruct((r, d), jnp.float32),
        scratch_types=[
            pltpu.VMEM((rows_per,), jnp.int32),
            pltpu.VMEM((rows_per, d), jnp.float32),
            pltpu.SemaphoreType.DMA,
        ],
    )
    def gk(table_hbm, idx_hbm, out_hbm, idx_v, rows_v, sem):
        wid = lax.axis_index("s") * nc + lax.axis_index("c")

        @pl.when(wid < nw_used)
        def _():
            base = wid * rows_per
            pltpu.sync_copy(idx_hbm.at[pl.ds(base, rows_per)], idx_v)
            pltpu.async_copy(table_hbm.at[idx_v], rows_v, sem).wait()
            pltpu.sync_copy(rows_v, out_hbm.at[pl.ds(base, rows_per)])

    return gk(table, idx)


def kernel(logits, emb, sent_mask, f_mean, g_mean, g2_mean, fg_mean):
    b, s, v = logits.shape
    d = emb.shape[1]
    r = b * s
    key_data = jax.random.key_data(
        jax.random.split(jax.random.key(42), 8)
    )[0].astype(jnp.uint32)
    gumbel2, hard2, idxb = _dense_stage(key_data, logits.reshape(r, v), rb=16)
    idx_flat = idxb[:, 0]
    f = _sc_gather(emb, idx_flat).reshape(b, s, d)
    return f, gumbel2.reshape(b, s, v), hard2.reshape(b, s, v)


# speedup vs baseline: 1.0312x; 1.0143x over previous
import functools

import jax
import jax.numpy as jnp
from jax import lax
from jax.experimental import pallas as pl
from jax.experimental.pallas import tpu as pltpu
from jax.experimental.pallas import tpu_sc as plsc

_ROT_A = (13, 15, 26, 6)
_ROT_B = (17, 29, 16, 24)
_BIG = 1e10


def _rotl(x, r):
    return (x << jnp.uint32(r)) | (x >> jnp.uint32(32 - r))


def _rounds(x0, x1, rots):
    for r in rots:
        x0 = x0 + x1
        x1 = _rotl(x1, r)
        x1 = x0 ^ x1
    return x0, x1


def _threefry_bits(k0, k1, x1):
    ks2 = k0 ^ k1 ^ jnp.uint32(0x1BD11BDA)
    x0 = x1 + k0
    x1 = _rotl(x1, _ROT_A[0])
    x1 = x0 ^ x1
    x0, x1 = _rounds(x0, x1, _ROT_A[1:])
    x0 = x0 + k1
    x1 = x1 + (ks2 + jnp.uint32(1))
    x0, x1 = _rounds(x0, x1, _ROT_B)
    x0 = x0 + ks2
    x1 = x1 + (k0 + jnp.uint32(2))
    x0, x1 = _rounds(x0, x1, _ROT_A)
    x0 = x0 + k0
    x1 = x1 + (k1 + jnp.uint32(3))
    x0, x1 = _rounds(x0, x1, _ROT_B)
    x0 = x0 + k1
    x1 = x1 + (ks2 + jnp.uint32(4))
    x0, x1 = _rounds(x0, x1, _ROT_A)
    x0 = x0 + ks2
    x1 = x1 + (k0 + jnp.uint32(5))
    return x0 ^ x1


def _gu_tile(logits_ref, gumbel_ref, k0, k1, rowk, c0, tl, rb):
    col_u = lax.broadcasted_iota(jnp.uint32, (rb, tl), 1)
    x1 = col_u + (rowk + jnp.uint32(c0) if isinstance(c0, int)
                  else rowk + c0.astype(jnp.uint32))
    bits = _threefry_bits(k0, k1, x1)
    fb = (bits >> jnp.uint32(9)) | jnp.uint32(0x3F800000)
    f = lax.bitcast_convert_type(fb, jnp.float32)
    e = -jnp.log(2.0 - f)
    gum = jnp.minimum(-jnp.log(e), _BIG)
    gu = logits_ref[:, pl.ds(c0, tl)] + gum
    gumbel_ref[:, pl.ds(c0, tl)] = gu
    return gu


_TILE = 8192


def _rebar_body(key_ref, logits_ref, gumbel_ref, hard_ref, idx_ref, *, rb, v):
    g = pl.program_id(0)
    k0 = key_ref[0]
    k1 = key_ref[1]
    nt = v // _TILE
    tail = v - nt * _TILE
    c_tail = nt * _TILE
    rowk = (
        lax.broadcasted_iota(jnp.uint32, (rb, 1), 0) * jnp.uint32(v)
        + ((g * (rb * v)).astype(jnp.uint32) + k1)
    )

    def step_a(j, macc):
        c0 = pl.multiple_of(j * _TILE, _TILE)
        gu = _gu_tile(logits_ref, gumbel_ref, k0, k1, rowk, c0, _TILE, rb)
        return jnp.maximum(macc, gu)

    macc = lax.fori_loop(
        0, nt, step_a, jnp.full((rb, _TILE), -jnp.inf, jnp.float32),
        unroll=2,
    )
    gu_t = _gu_tile(logits_ref, gumbel_ref, k0, k1, rowk, c_tail, tail, rb)
    m = jnp.maximum(
        jnp.max(macc, axis=1, keepdims=True),
        jnp.max(gu_t, axis=1, keepdims=True),
    )

    def step_b(j, carry):
        sacc, iacc = carry
        c0 = pl.multiple_of(j * _TILE, _TILE)
        gu = gumbel_ref[:, pl.ds(c0, _TILE)]
        sacc = sacc + jnp.exp(gu - m)
        col = lax.broadcasted_iota(jnp.int32, (rb, _TILE), 1) + c0
        iacc = jnp.minimum(iacc, jnp.where(gu >= m, col, v))
        return sacc, iacc

    sacc, iacc = lax.fori_loop(
        0,
        nt,
        step_b,
        (
            jnp.zeros((rb, _TILE), jnp.float32),
            jnp.full((rb, _TILE), v, jnp.int32),
        ),
        unroll=2,
    )
    p_t = jnp.exp(gu_t - m)
    col_t = lax.broadcasted_iota(jnp.int32, (rb, tail), 1) + c_tail
    s = (
        jnp.sum(sacc, axis=1, keepdims=True)
        + jnp.sum(p_t, axis=1, keepdims=True)
    )
    idx = jnp.minimum(
        jnp.min(iacc, axis=1, keepdims=True),
        jnp.min(jnp.where(gu_t >= m, col_t, v), axis=1, keepdims=True),
    )
    d = m + jnp.log(s)

    def step_c(j, carry):
        c0 = pl.multiple_of(j * _TILE, _TILE)
        gu = gumbel_ref[:, pl.ds(c0, _TILE)]
        gumbel_ref[:, pl.ds(c0, _TILE)] = jnp.exp(gu - d)
        col = lax.broadcasted_iota(jnp.int32, (rb, _TILE), 1) + c0
        hard_ref[:, pl.ds(c0, _TILE)] = jnp.where(col == idx, 1.0, 0.0)
        return carry

    lax.fori_loop(0, nt, step_c, 0, unroll=2)
    gumbel_ref[:, pl.ds(c_tail, tail)] = jnp.exp(gu_t - d)
    hard_ref[:, pl.ds(c_tail, tail)] = jnp.where(col_t == idx, 1.0, 0.0)
    idx_ref[...] = jnp.broadcast_to(idx, (rb, 128))


def _dense_stage(key_data, logits2, rb):
    r, v = logits2.shape
    return pl.pallas_call(
        functools.partial(_rebar_body, rb=rb, v=v),
        grid=(r // rb,),
        in_specs=[
            pl.BlockSpec(memory_space=pltpu.SMEM),
            pl.BlockSpec((rb, v), lambda g: (g, 0)),
        ],
        out_specs=[
            pl.BlockSpec((rb, v), lambda g: (g, 0)),
            pl.BlockSpec((rb, v), lambda g: (g, 0)),
            pl.BlockSpec((rb, 128), lambda g: (g, 0)),
        ],
        out_shape=[
            jax.ShapeDtypeStruct((r, v), jnp.float32),
            jax.ShapeDtypeStruct((r, v), jnp.float32),
            jax.ShapeDtypeStruct((r, 128), jnp.int32),
        ],
        compiler_params=pltpu.CompilerParams(
            dimension_semantics=("parallel",),
        ),
    )(key_data, logits2)


def _sc_gather(table, idx):
    r = idx.shape[0]
    d = table.shape[1]
    info = plsc.get_sparse_core_info()
    nc = info.num_cores
    nw_used = 16
    rows_per = r // nw_used
    mesh = plsc.VectorSubcoreMesh(core_axis_name="c", subcore_axis_name="s")

    @functools.partial(
        pl.kernel,
        mesh=mesh,
        out_type=jax.ShapeDtypeStruct((r, d), jnp.float32),
        scratch_types=[
            pltpu.VMEM((rows_per,), jnp.int32),
            pltpu.VMEM((rows_per, d), jnp.float32),
            pltpu.SemaphoreType.DMA,
        ],
    )
    def gk(table_hbm, idx_hbm, out_hbm, idx_v, rows_v, sem):
        wid = lax.axis_index("s") * nc + lax.axis_index("c")

        @pl.when(wid < nw_used)
        def _():
            base = wid * rows_per
            pltpu.sync_copy(idx_hbm.at[pl.ds(base, rows_per)], idx_v)
            pltpu.async_copy(table_hbm.at[idx_v], rows_v, sem).wait()
            pltpu.sync_copy(rows_v, out_hbm.at[pl.ds(base, rows_per)])

    return gk(table, idx)


def kernel(logits, emb, sent_mask, f_mean, g_mean, g2_mean, fg_mean):
    b, s, v = logits.shape
    d = emb.shape[1]
    r = b * s
    key_data = jax.random.key_data(
        jax.random.split(jax.random.key(42), 8)
    )[0].astype(jnp.uint32)
    gumbel2, hard2, idxb = _dense_stage(key_data, logits.reshape(r, v), rb=16)
    idx_flat = idxb[:, 0]
    f = _sc_gather(emb, idx_flat).reshape(b, s, d)
    return f, gumbel2.reshape(b, s, v), hard2.reshape(b, s, v)


# speedup vs baseline: 1.0471x; 1.0154x over previous
import functools

import jax
import jax.numpy as jnp
from jax import lax
from jax.experimental import pallas as pl
from jax.experimental.pallas import tpu as pltpu
from jax.experimental.pallas import tpu_sc as plsc

_ROT_A = (13, 15, 26, 6)
_ROT_B = (17, 29, 16, 24)
_BIG = 1e10


def _rotl(x, r):
    return (x << jnp.uint32(r)) | (x >> jnp.uint32(32 - r))


def _rounds(x0, x1, rots):
    for r in rots:
        x0 = x0 + x1
        x1 = _rotl(x1, r)
        x1 = x0 ^ x1
    return x0, x1


def _threefry_bits(k0, k1, x1):
    ks2 = k0 ^ k1 ^ jnp.uint32(0x1BD11BDA)
    x0 = x1 + k0
    x1 = _rotl(x1, _ROT_A[0])
    x1 = x0 ^ x1
    x0, x1 = _rounds(x0, x1, _ROT_A[1:])
    x0 = x0 + k1
    x1 = x1 + (ks2 + jnp.uint32(1))
    x0, x1 = _rounds(x0, x1, _ROT_B)
    x0 = x0 + ks2
    x1 = x1 + (k0 + jnp.uint32(2))
    x0, x1 = _rounds(x0, x1, _ROT_A)
    x0 = x0 + k0
    x1 = x1 + (k1 + jnp.uint32(3))
    x0, x1 = _rounds(x0, x1, _ROT_B)
    x0 = x0 + k1
    x1 = x1 + (ks2 + jnp.uint32(4))
    x0, x1 = _rounds(x0, x1, _ROT_A)
    x0 = x0 + ks2
    x1 = x1 + (k0 + jnp.uint32(5))
    return x0 ^ x1


def _gu_tile(logits_ref, gumbel_ref, k0, k1, rowk, c0, tl, rb):
    col_u = lax.broadcasted_iota(jnp.uint32, (rb, tl), 1)
    x1 = col_u + (rowk + jnp.uint32(c0) if isinstance(c0, int)
                  else rowk + c0.astype(jnp.uint32))
    bits = _threefry_bits(k0, k1, x1)
    fb = (bits >> jnp.uint32(9)) | jnp.uint32(0x3F800000)
    f = lax.bitcast_convert_type(fb, jnp.float32)
    e = -jnp.log(2.0 - f)
    gum = jnp.minimum(-jnp.log(e), _BIG)
    gu = logits_ref[:, pl.ds(c0, tl)] + gum
    gumbel_ref[:, pl.ds(c0, tl)] = gu
    return gu


_TILE = 8192


def _rebar_body(key_ref, logits_ref, gumbel_ref, hard_ref, idx_ref, *, rb, v):
    g = pl.program_id(0)
    k0 = key_ref[0]
    k1 = key_ref[1]
    nt = v // _TILE
    tail = v - nt * _TILE
    c_tail = nt * _TILE
    rowk = (
        lax.broadcasted_iota(jnp.uint32, (rb, 1), 0) * jnp.uint32(v)
        + ((g * (rb * v)).astype(jnp.uint32) + k1)
    )

    def step_a(j, macc):
        c0 = pl.multiple_of(j * _TILE, _TILE)
        gu = _gu_tile(logits_ref, gumbel_ref, k0, k1, rowk, c0, _TILE, rb)
        return jnp.maximum(macc, gu)

    macc = lax.fori_loop(
        0, nt, step_a, jnp.full((rb, _TILE), -jnp.inf, jnp.float32),
        unroll=4,
    )
    gu_t = _gu_tile(logits_ref, gumbel_ref, k0, k1, rowk, c_tail, tail, rb)
    m = jnp.maximum(
        jnp.max(macc, axis=1, keepdims=True),
        jnp.max(gu_t, axis=1, keepdims=True),
    )

    def step_b(j, carry):
        sacc, iacc = carry
        c0 = pl.multiple_of(j * _TILE, _TILE)
        gu = gumbel_ref[:, pl.ds(c0, _TILE)]
        sacc = sacc + jnp.exp(gu - m)
        col = lax.broadcasted_iota(jnp.int32, (rb, _TILE), 1) + c0
        iacc = jnp.minimum(iacc, jnp.where(gu >= m, col, v))
        return sacc, iacc

    sacc, iacc = lax.fori_loop(
        0,
        nt,
        step_b,
        (
            jnp.zeros((rb, _TILE), jnp.float32),
            jnp.full((rb, _TILE), v, jnp.int32),
        ),
        unroll=4,
    )
    p_t = jnp.exp(gu_t - m)
    col_t = lax.broadcasted_iota(jnp.int32, (rb, tail), 1) + c_tail
    s = (
        jnp.sum(sacc, axis=1, keepdims=True)
        + jnp.sum(p_t, axis=1, keepdims=True)
    )
    idx = jnp.minimum(
        jnp.min(iacc, axis=1, keepdims=True),
        jnp.min(jnp.where(gu_t >= m, col_t, v), axis=1, keepdims=True),
    )
    d = m + jnp.log(s)

    def step_c(j, carry):
        c0 = pl.multiple_of(j * _TILE, _TILE)
        gu = gumbel_ref[:, pl.ds(c0, _TILE)]
        gumbel_ref[:, pl.ds(c0, _TILE)] = jnp.exp(gu - d)
        col = lax.broadcasted_iota(jnp.int32, (rb, _TILE), 1) + c0
        hard_ref[:, pl.ds(c0, _TILE)] = jnp.where(col == idx, 1.0, 0.0)
        return carry

    lax.fori_loop(0, nt, step_c, 0, unroll=4)
    gumbel_ref[:, pl.ds(c_tail, tail)] = jnp.exp(gu_t - d)
    hard_ref[:, pl.ds(c_tail, tail)] = jnp.where(col_t == idx, 1.0, 0.0)
    idx_ref[...] = jnp.broadcast_to(idx, (rb, 128))


def _dense_stage(key_data, logits2, rb):
    r, v = logits2.shape
    return pl.pallas_call(
        functools.partial(_rebar_body, rb=rb, v=v),
        grid=(r // rb,),
        in_specs=[
            pl.BlockSpec(memory_space=pltpu.SMEM),
            pl.BlockSpec((rb, v), lambda g: (g, 0)),
        ],
        out_specs=[
            pl.BlockSpec((rb, v), lambda g: (g, 0)),
            pl.BlockSpec((rb, v), lambda g: (g, 0)),
            pl.BlockSpec((rb, 128), lambda g: (g, 0)),
        ],
        out_shape=[
            jax.ShapeDtypeStruct((r, v), jnp.float32),
            jax.ShapeDtypeStruct((r, v), jnp.float32),
            jax.ShapeDtypeStruct((r, 128), jnp.int32),
        ],
        compiler_params=pltpu.CompilerParams(
            dimension_semantics=("parallel",),
        ),
    )(key_data, logits2)


def _sc_gather(table, idx):
    r = idx.shape[0]
    d = table.shape[1]
    info = plsc.get_sparse_core_info()
    nc = info.num_cores
    nw_used = 16
    rows_per = r // nw_used
    mesh = plsc.VectorSubcoreMesh(core_axis_name="c", subcore_axis_name="s")

    @functools.partial(
        pl.kernel,
        mesh=mesh,
        out_type=jax.ShapeDtypeStruct((r, d), jnp.float32),
        scratch_types=[
            pltpu.VMEM((rows_per,), jnp.int32),
            pltpu.VMEM((rows_per, d), jnp.float32),
            pltpu.SemaphoreType.DMA,
        ],
    )
    def gk(table_hbm, idx_hbm, out_hbm, idx_v, rows_v, sem):
        wid = lax.axis_index("s") * nc + lax.axis_index("c")

        @pl.when(wid < nw_used)
        def _():
            base = wid * rows_per
            pltpu.sync_copy(idx_hbm.at[pl.ds(base, rows_per)], idx_v)
            pltpu.async_copy(table_hbm.at[idx_v], rows_v, sem).wait()
            pltpu.sync_copy(rows_v, out_hbm.at[pl.ds(base, rows_per)])

    return gk(table, idx)


def kernel(logits, emb, sent_mask, f_mean, g_mean, g2_mean, fg_mean):
    b, s, v = logits.shape
    d = emb.shape[1]
    r = b * s
    key_data = jax.random.key_data(
        jax.random.split(jax.random.key(42), 8)
    )[0].astype(jnp.uint32)
    gumbel2, hard2, idxb = _dense_stage(key_data, logits.reshape(r, v), rb=16)
    idx_flat = idxb[:, 0]
    f = _sc_gather(emb, idx_flat).reshape(b, s, d)
    return f, gumbel2.reshape(b, s, v), hard2.reshape(b, s, v)


# speedup vs baseline: 1.0541x; 1.0067x over previous
import functools

import jax
import jax.numpy as jnp
from jax import lax
from jax.experimental import pallas as pl
from jax.experimental.pallas import tpu as pltpu
from jax.experimental.pallas import tpu_sc as plsc

_ROT_A = (13, 15, 26, 6)
_ROT_B = (17, 29, 16, 24)
_BIG = 1e10


def _rotl(x, r):
    return (x << jnp.uint32(r)) | (x >> jnp.uint32(32 - r))


def _rounds(x0, x1, rots):
    for r in rots:
        x0 = x0 + x1
        x1 = _rotl(x1, r)
        x1 = x0 ^ x1
    return x0, x1


def _threefry_bits(k0, k1, x1):
    ks2 = k0 ^ k1 ^ jnp.uint32(0x1BD11BDA)
    x0 = x1 + k0
    x1 = _rotl(x1, _ROT_A[0])
    x1 = x0 ^ x1
    x0, x1 = _rounds(x0, x1, _ROT_A[1:])
    x0 = x0 + k1
    x1 = x1 + (ks2 + jnp.uint32(1))
    x0, x1 = _rounds(x0, x1, _ROT_B)
    x0 = x0 + ks2
    x1 = x1 + (k0 + jnp.uint32(2))
    x0, x1 = _rounds(x0, x1, _ROT_A)
    x0 = x0 + k0
    x1 = x1 + (k1 + jnp.uint32(3))
    x0, x1 = _rounds(x0, x1, _ROT_B)
    x0 = x0 + k1
    x1 = x1 + (ks2 + jnp.uint32(4))
    x0, x1 = _rounds(x0, x1, _ROT_A)
    x0 = x0 + ks2
    x1 = x1 + (k0 + jnp.uint32(5))
    return x0 ^ x1


def _gu_tile(logits_ref, gumbel_ref, k0, k1, rowk, c0, tl, rb):
    col_u = lax.broadcasted_iota(jnp.uint32, (rb, tl), 1)
    x1 = col_u + (rowk + jnp.uint32(c0) if isinstance(c0, int)
                  else rowk + c0.astype(jnp.uint32))
    bits = _threefry_bits(k0, k1, x1)
    fb = (bits >> jnp.uint32(9)) | jnp.uint32(0x3F800000)
    f = lax.bitcast_convert_type(fb, jnp.float32)
    e = -jnp.log(2.0 - f)
    gum = jnp.minimum(-jnp.log(e), _BIG)
    gu = logits_ref[:, pl.ds(c0, tl)] + gum
    gumbel_ref[:, pl.ds(c0, tl)] = gu
    return gu


_TILE = 8192


def _rebar_body(key_ref, logits_ref, gumbel_ref, hard_ref, idx_ref, *, rb, v):
    g = pl.program_id(0)
    k0 = key_ref[0]
    k1 = key_ref[1]
    nt = v // _TILE
    tail = v - nt * _TILE
    c_tail = nt * _TILE
    rowk = (
        lax.broadcasted_iota(jnp.uint32, (rb, 1), 0) * jnp.uint32(v)
        + ((g * (rb * v)).astype(jnp.uint32) + k1)
    )

    def step_a(j, macc):
        c0 = pl.multiple_of(j * _TILE, _TILE)
        gu = _gu_tile(logits_ref, gumbel_ref, k0, k1, rowk, c0, _TILE, rb)
        return jnp.maximum(macc, gu)

    macc = lax.fori_loop(
        0, nt, step_a, jnp.full((rb, _TILE), -jnp.inf, jnp.float32),
        unroll=6,
    )
    gu_t = _gu_tile(logits_ref, gumbel_ref, k0, k1, rowk, c_tail, tail, rb)
    m = jnp.maximum(
        jnp.max(macc, axis=1, keepdims=True),
        jnp.max(gu_t, axis=1, keepdims=True),
    )

    def step_b(j, carry):
        sacc, iacc = carry
        c0 = pl.multiple_of(j * _TILE, _TILE)
        gu = gumbel_ref[:, pl.ds(c0, _TILE)]
        sacc = sacc + jnp.exp(gu - m)
        col = lax.broadcasted_iota(jnp.int32, (rb, _TILE), 1) + c0
        iacc = jnp.minimum(iacc, jnp.where(gu >= m, col, v))
        return sacc, iacc

    sacc, iacc = lax.fori_loop(
        0,
        nt,
        step_b,
        (
            jnp.zeros((rb, _TILE), jnp.float32),
            jnp.full((rb, _TILE), v, jnp.int32),
        ),
        unroll=6,
    )
    p_t = jnp.exp(gu_t - m)
    col_t = lax.broadcasted_iota(jnp.int32, (rb, tail), 1) + c_tail
    s = (
        jnp.sum(sacc, axis=1, keepdims=True)
        + jnp.sum(p_t, axis=1, keepdims=True)
    )
    idx = jnp.minimum(
        jnp.min(iacc, axis=1, keepdims=True),
        jnp.min(jnp.where(gu_t >= m, col_t, v), axis=1, keepdims=True),
    )
    d = m + jnp.log(s)

    def step_c(j, carry):
        c0 = pl.multiple_of(j * _TILE, _TILE)
        gu = gumbel_ref[:, pl.ds(c0, _TILE)]
        gumbel_ref[:, pl.ds(c0, _TILE)] = jnp.exp(gu - d)
        col = lax.broadcasted_iota(jnp.int32, (rb, _TILE), 1) + c0
        hard_ref[:, pl.ds(c0, _TILE)] = jnp.where(col == idx, 1.0, 0.0)
        return carry

    lax.fori_loop(0, nt, step_c, 0, unroll=6)
    gumbel_ref[:, pl.ds(c_tail, tail)] = jnp.exp(gu_t - d)
    hard_ref[:, pl.ds(c_tail, tail)] = jnp.where(col_t == idx, 1.0, 0.0)
    idx_ref[...] = jnp.broadcast_to(idx, (rb, 128))


def _dense_stage(key_data, logits2, rb):
    r, v = logits2.shape
    return pl.pallas_call(
        functools.partial(_rebar_body, rb=rb, v=v),
        grid=(r // rb,),
        in_specs=[
            pl.BlockSpec(memory_space=pltpu.SMEM),
            pl.BlockSpec((rb, v), lambda g: (g, 0)),
        ],
        out_specs=[
            pl.BlockSpec((rb, v), lambda g: (g, 0)),
            pl.BlockSpec((rb, v), lambda g: (g, 0)),
            pl.BlockSpec((rb, 128), lambda g: (g, 0)),
        ],
        out_shape=[
            jax.ShapeDtypeStruct((r, v), jnp.float32),
            jax.ShapeDtypeStruct((r, v), jnp.float32),
            jax.ShapeDtypeStruct((r, 128), jnp.int32),
        ],
        compiler_params=pltpu.CompilerParams(
            dimension_semantics=("parallel",),
        ),
    )(key_data, logits2)


def _sc_gather(table, idx):
    r = idx.shape[0]
    d = table.shape[1]
    info = plsc.get_sparse_core_info()
    nc = info.num_cores
    nw_used = 16
    rows_per = r // nw_used
    mesh = plsc.VectorSubcoreMesh(core_axis_name="c", subcore_axis_name="s")

    @functools.partial(
        pl.kernel,
        mesh=mesh,
        out_type=jax.ShapeDtypeStruct((r, d), jnp.float32),
        scratch_types=[
            pltpu.VMEM((rows_per,), jnp.int32),
            pltpu.VMEM((rows_per, d), jnp.float32),
            pltpu.SemaphoreType.DMA,
        ],
    )
    def gk(table_hbm, idx_hbm, out_hbm, idx_v, rows_v, sem):
        wid = lax.axis_index("s") * nc + lax.axis_index("c")

        @pl.when(wid < nw_used)
        def _():
            base = wid * rows_per
            pltpu.sync_copy(idx_hbm.at[pl.ds(base, rows_per)], idx_v)
            pltpu.async_copy(table_hbm.at[idx_v], rows_v, sem).wait()
            pltpu.sync_copy(rows_v, out_hbm.at[pl.ds(base, rows_per)])

    return gk(table, idx)


def kernel(logits, emb, sent_mask, f_mean, g_mean, g2_mean, fg_mean):
    b, s, v = logits.shape
    d = emb.shape[1]
    r = b * s
    key_data = jax.random.key_data(
        jax.random.split(jax.random.key(42), 8)
    )[0].astype(jnp.uint32)
    gumbel2, hard2, idxb = _dense_stage(key_data, logits.reshape(r, v), rb=16)
    idx_flat = idxb[:, 0]
    f = _sc_gather(emb, idx_flat).reshape(b, s, d)
    return f, gumbel2.reshape(b, s, v), hard2.reshape(b, s, v)
